# baseline (speedup 1.0000x reference)
import jax
import jax.numpy as jnp
from jax.experimental import pallas as pl
from jax.experimental.pallas import tpu as pltpu

_N_CHUNKS = 16


def kernel(x, position_embeddings):
    seq_len = x.shape[1]
    d_model = position_embeddings.shape[1]
    chunk = seq_len // _N_CHUNKS

    def body(in_hbm, out_hbm, scratch, isem, osem):
        ins = []
        for i in range(_N_CHUNKS):
            ins.append(pltpu.make_async_copy(
                in_hbm.at[pl.ds(i * chunk, chunk), :],
                scratch.at[i], isem.at[i]))
            ins[i].start()
        outs = []
        for i in range(_N_CHUNKS):
            ins[i].wait()
            outs.append(pltpu.make_async_copy(
                scratch.at[i],
                out_hbm.at[pl.ds(i * chunk, chunk), :], osem.at[i]))
            outs[i].start()
        for i in range(_N_CHUNKS):
            outs[i].wait()

    out = pl.pallas_call(
        body,
        in_specs=[pl.BlockSpec(memory_space=pl.ANY)],
        out_specs=pl.BlockSpec(memory_space=pl.ANY),
        out_shape=jax.ShapeDtypeStruct((seq_len, d_model), position_embeddings.dtype),
        scratch_shapes=[
            pltpu.VMEM((_N_CHUNKS, chunk, d_model), jnp.float32),
            pltpu.SemaphoreType.DMA((_N_CHUNKS,)),
            pltpu.SemaphoreType.DMA((_N_CHUNKS,)),
        ],
    )(position_embeddings)
    return out[None, :, :]


# speedup vs baseline: 1.0238x; 1.0238x over previous
import jax
import jax.numpy as jnp
from jax.experimental import pallas as pl
from jax.experimental.pallas import tpu as pltpu

_N_CHUNKS = 4


def kernel(x, position_embeddings):
    seq_len = x.shape[1]
    d_model = position_embeddings.shape[1]
    chunk = seq_len // _N_CHUNKS

    def body(in_hbm, out_hbm, scratch, isem, osem):
        ins = []
        for i in range(_N_CHUNKS):
            ins.append(pltpu.make_async_copy(
                in_hbm.at[pl.ds(i * chunk, chunk), :],
                scratch.at[i], isem.at[i]))
            ins[i].start()
        outs = []
        for i in range(_N_CHUNKS):
            ins[i].wait()
            outs.append(pltpu.make_async_copy(
                scratch.at[i],
                out_hbm.at[pl.ds(i * chunk, chunk), :], osem.at[i]))
            outs[i].start()
        for i in range(_N_CHUNKS):
            outs[i].wait()

    out = pl.pallas_call(
        body,
        in_specs=[pl.BlockSpec(memory_space=pl.ANY)],
        out_specs=pl.BlockSpec(memory_space=pl.ANY),
        out_shape=jax.ShapeDtypeStruct((seq_len, d_model), position_embeddings.dtype),
        scratch_shapes=[
            pltpu.VMEM((_N_CHUNKS, chunk, d_model), jnp.float32),
            pltpu.SemaphoreType.DMA((_N_CHUNKS,)),
            pltpu.SemaphoreType.DMA((_N_CHUNKS,)),
        ],
    )(position_embeddings)
    return out[None, :, :]


# speedup vs baseline: 1.0310x; 1.0070x over previous
import jax
import jax.numpy as jnp
from jax.experimental import pallas as pl
from jax.experimental.pallas import tpu as pltpu


def _copy_block(in_ref, o_ref):
    o_ref[...] = in_ref[...]


def kernel(x, position_embeddings):
    seq_len = x.shape[1]
    d_model = position_embeddings.shape[1]
    block = 1024
    out = pl.pallas_call(
        _copy_block,
        grid=(seq_len // block,),
        in_specs=[pl.BlockSpec((block, d_model), lambda i: (i, 0))],
        out_specs=pl.BlockSpec((block, d_model), lambda i: (i, 0)),
        out_shape=jax.ShapeDtypeStruct((seq_len, d_model), position_embeddings.dtype),
        compiler_params=pltpu.CompilerParams(
            dimension_semantics=("parallel",),
        ),
    )(position_embeddings)
    return out[None, :, :]
